# Initial kernel scaffold; baseline (speedup 1.0000x reference)
import jax
import jax.numpy as jnp
from jax.experimental import pallas as pl


def kernel(memory, write_content_key, write_beta, erase_vector, write_vector, alloc_gate, write_gate, free_gates, prev_read_dist):
    raise NotImplementedError("write your pallas kernel here")


# speedup vs baseline: 4.7225x; 4.7225x over previous
import numpy as np
import jax
import jax.numpy as jnp
from jax.experimental import pallas as pl

_EPS = 1e-06


def _alloc_const(n):
    u = np.full((n,), _EPS, dtype=np.float32)
    u_prod = np.cumprod(u, dtype=np.float32)
    one_minus = (np.float32(1.0) - u).astype(np.float32)
    scores = np.concatenate([one_minus[:1], one_minus[1:] * u_prod[:-1]])
    return scores.astype(np.float32)


def _body(mem_ref, key_ref, beta_ref, erase_ref, wv_ref, ag_ref, wg_ref,
          fg_ref, prd_ref, alloc_ref, out_ref):
    b = pl.program_id(0)
    mem = mem_ref[0]
    key = key_ref[b]
    beta = beta_ref[b, 0]
    ag = ag_ref[b, 0]
    wg = wg_ref[b, 0]

    dot = jnp.sum(mem * key[None, :], axis=1, keepdims=True)
    msq = jnp.sum(mem * mem, axis=1, keepdims=True)
    knorm = jnp.sqrt(jnp.sum(key * key))
    norm = knorm * jnp.sqrt(msq)
    score = beta * (dot / (norm + _EPS))
    score = score - jnp.max(score)
    e = jnp.exp(score)
    content = e / jnp.sum(e)

    prd = prd_ref[0]
    R = prd.shape[0]
    phi = 1.0 - fg_ref[b, 0] * prd[0]
    for r in range(1, R):
        phi = phi * (1.0 - fg_ref[b, r] * prd[r])
    phi = phi[:, None]

    alloc = alloc_ref[0][:, None]
    wd = wg * (ag * alloc + (1.0 - ag) * content)

    erase = erase_ref[b][None, :]
    wv = wv_ref[b][None, :]
    out_ref[0] = mem * ((1.0 - wd * erase) * phi) + wd * wv


def kernel(memory, write_content_key, write_beta, erase_vector, write_vector,
           alloc_gate, write_gate, free_gates, prev_read_dist):
    B, N, W = memory.shape
    alloc = jnp.asarray(_alloc_const(N))[None, :]

    full = lambda a: pl.BlockSpec(a.shape, lambda b: (0,) * a.ndim)
    return pl.pallas_call(
        _body,
        grid=(B,),
        in_specs=[
            pl.BlockSpec((1, N, W), lambda b: (b, 0, 0)),
            full(write_content_key),
            full(write_beta),
            full(erase_vector),
            full(write_vector),
            full(alloc_gate),
            full(write_gate),
            full(free_gates),
            pl.BlockSpec((1,) + prev_read_dist.shape[1:], lambda b: (b, 0, 0)),
            full(alloc),
        ],
        out_specs=pl.BlockSpec((1, N, W), lambda b: (b, 0, 0)),
        out_shape=jax.ShapeDtypeStruct((B, N, W), memory.dtype),
    )(memory, write_content_key, write_beta, erase_vector, write_vector,
      alloc_gate, write_gate, free_gates, prev_read_dist, alloc)


# speedup vs baseline: 5.9676x; 1.2637x over previous
import numpy as np
import jax
import jax.numpy as jnp
from jax.experimental import pallas as pl

_EPS = 1e-06


def _alloc_const(n):
    u = np.full((n,), _EPS, dtype=np.float32)
    u_prod = np.cumprod(u, dtype=np.float32)
    one_minus = (np.float32(1.0) - u).astype(np.float32)
    scores = np.concatenate([one_minus[:1], one_minus[1:] * u_prod[:-1]])
    return scores.astype(np.float32)


def _body(mem_ref, key_ref, beta_ref, erase_ref, wv_ref, ag_ref, wg_ref,
          fg_ref, prd_ref, alloc_ref, out_ref):
    b = pl.program_id(0)
    mem = mem_ref[0]
    N, W = mem.shape
    S = N // 128
    beta = beta_ref[b, 0]
    ag = ag_ref[b, 0]
    wg = wg_ref[b, 0]

    keyrow = key_ref[b][None, :]
    onesrow = jnp.ones((1, W), dtype=mem.dtype)
    dot2 = jax.lax.dot_general(keyrow, mem, (((1,), (1,)), ((), ())),
                               preferred_element_type=jnp.float32)
    sq = mem * mem
    msq2 = jax.lax.dot_general(onesrow, sq, (((1,), (1,)), ((), ())),
                               preferred_element_type=jnp.float32)

    knorm = jnp.sqrt(jnp.sum(key_ref[b] * key_ref[b]))
    norm = knorm * jnp.sqrt(msq2)
    score = beta * (dot2 / (norm + _EPS))
    score = score - jnp.max(score)
    e = jnp.exp(score)
    content = e / jnp.sum(e)

    prd = prd_ref[0]
    R = prd.shape[0]
    phi = 1.0 - fg_ref[b, 0] * prd[0:1]
    for r in range(1, R):
        phi = phi * (1.0 - fg_ref[b, r] * prd[r:r + 1])

    alloc = alloc_ref[0][None, :]
    wd = wg * (ag * alloc + (1.0 - ag) * content)

    wd1 = wd.T
    phi1 = phi.T
    erase = erase_ref[b][None, :]
    wv = wv_ref[b][None, :]
    out_ref[0] = mem * ((1.0 - wd1 * erase) * phi1) + wd1 * wv


def kernel(memory, write_content_key, write_beta, erase_vector, write_vector,
           alloc_gate, write_gate, free_gates, prev_read_dist):
    B, N, W = memory.shape
    R = free_gates.shape[1]
    alloc = jnp.asarray(_alloc_const(N))[None, :]
    prd = prev_read_dist

    full = lambda a: pl.BlockSpec(a.shape, lambda b: (0,) * a.ndim)
    return pl.pallas_call(
        _body,
        grid=(B,),
        in_specs=[
            pl.BlockSpec((1, N, W), lambda b: (b, 0, 0)),
            full(write_content_key),
            full(write_beta),
            full(erase_vector),
            full(write_vector),
            full(alloc_gate),
            full(write_gate),
            full(free_gates),
            pl.BlockSpec((1, R, N), lambda b: (b, 0, 0)),
            full(alloc),
        ],
        out_specs=pl.BlockSpec((1, N, W), lambda b: (b, 0, 0)),
        out_shape=jax.ShapeDtypeStruct((B, N, W), memory.dtype),
    )(memory, write_content_key, write_beta, erase_vector, write_vector,
      alloc_gate, write_gate, free_gates, prd, alloc)


# speedup vs baseline: 8.8476x; 1.4826x over previous
import numpy as np
import jax
import jax.numpy as jnp
from jax.experimental import pallas as pl

_EPS = 1e-06


def _alloc_const(n):
    u = np.full((n,), _EPS, dtype=np.float32)
    u_prod = np.cumprod(u, dtype=np.float32)
    one_minus = (np.float32(1.0) - u).astype(np.float32)
    scores = np.concatenate([one_minus[:1], one_minus[1:] * u_prod[:-1]])
    return scores.astype(np.float32)


def _body(mem_ref, key_ref, beta_ref, erase_ref, wv_ref, ag_ref, wg_ref,
          fg_ref, prd_ref, alloc_ref, out_ref):
    b = pl.program_id(0)
    mem = mem_ref[0]
    N, W = mem.shape
    beta = beta_ref[b, 0]
    ag = ag_ref[b, 0]
    wg = wg_ref[b, 0]

    keyrow = key_ref[b][None, :]
    onesrow = jnp.ones((1, W), dtype=mem.dtype)
    dot2 = jax.lax.dot_general(keyrow, mem, (((1,), (1,)), ((), ())),
                               preferred_element_type=jnp.float32)
    sq = mem * mem
    msq2 = jax.lax.dot_general(onesrow, sq, (((1,), (1,)), ((), ())),
                               preferred_element_type=jnp.float32)

    knorm = jnp.sqrt(jnp.sum(key_ref[b] * key_ref[b]))
    norm = knorm * jnp.sqrt(msq2)
    score = beta * (dot2 / (norm + _EPS))
    e = jnp.exp(score)
    inv_se = 1.0 / jnp.sum(e)

    prd = prd_ref[0]
    R = prd.shape[0]
    phi = 1.0 - fg_ref[b, 0] * prd[0:1]
    for r in range(1, R):
        phi = phi * (1.0 - fg_ref[b, r] * prd[r:r + 1])

    alloc = alloc_ref[0][None, :]
    wd = (wg * ag) * alloc + (wg * (1.0 - ag) * inv_se) * e

    erase = erase_ref[b][None, :]
    wv = wv_ref[b][None, :]
    lhs = jnp.concatenate([phi, -wd * phi], axis=0)
    rhs = jnp.concatenate([onesrow, erase], axis=0)
    m1 = jax.lax.dot_general(lhs, rhs, (((0,), (0,)), ((), ())),
                             preferred_element_type=jnp.float32)
    o2 = jax.lax.dot_general(wd, wv, (((0,), (0,)), ((), ())),
                             preferred_element_type=jnp.float32)
    out_ref[0] = mem * m1 + o2


def kernel(memory, write_content_key, write_beta, erase_vector, write_vector,
           alloc_gate, write_gate, free_gates, prev_read_dist):
    B, N, W = memory.shape
    R = free_gates.shape[1]
    alloc = jnp.asarray(_alloc_const(N))[None, :]

    full = lambda a: pl.BlockSpec(a.shape, lambda b: (0,) * a.ndim)
    return pl.pallas_call(
        _body,
        grid=(B,),
        in_specs=[
            pl.BlockSpec((1, N, W), lambda b: (b, 0, 0)),
            full(write_content_key),
            full(write_beta),
            full(erase_vector),
            full(write_vector),
            full(alloc_gate),
            full(write_gate),
            full(free_gates),
            pl.BlockSpec((1, R, N), lambda b: (b, 0, 0)),
            full(alloc),
        ],
        out_specs=pl.BlockSpec((1, N, W), lambda b: (b, 0, 0)),
        out_shape=jax.ShapeDtypeStruct((B, N, W), memory.dtype),
    )(memory, write_content_key, write_beta, erase_vector, write_vector,
      alloc_gate, write_gate, free_gates, prev_read_dist, alloc)


# speedup vs baseline: 8.8719x; 1.0027x over previous
import numpy as np
import jax
import jax.numpy as jnp
from jax.experimental import pallas as pl
from jax.experimental.pallas import tpu as pltpu

_EPS = 1e-06
_C = 8


def _alloc_const(n):
    u = np.full((n,), _EPS, dtype=np.float32)
    u_prod = np.cumprod(u, dtype=np.float32)
    one_minus = (np.float32(1.0) - u).astype(np.float32)
    scores = np.concatenate([one_minus[:1], one_minus[1:] * u_prod[:-1]])
    return scores.astype(np.float32)


def _body(mem_hbm, key_ref, beta_ref, erase_ref, wv_ref, ag_ref, wg_ref,
          fg_ref, prd_ref, alloc_ref, out_hbm,
          mem_v, out_v, dot_s, msq_s, in_sem, out_sem):
    b = pl.program_id(0)
    B = pl.num_programs(0)
    N, W = mem_hbm.shape[1], mem_hbm.shape[2]
    NC = N // _C
    p = jax.lax.rem(b, 2)

    def in_copy(batch, par, c):
        return pltpu.make_async_copy(
            mem_hbm.at[batch, pl.ds(c * NC, NC), :],
            mem_v.at[par, pl.ds(c * NC, NC), :],
            in_sem.at[par, c])

    def out_copy(batch, par, c):
        return pltpu.make_async_copy(
            out_v.at[par, pl.ds(c * NC, NC), :],
            out_hbm.at[batch, pl.ds(c * NC, NC), :],
            out_sem.at[par, c])

    @pl.when(b == 0)
    def _():
        for c in range(_C):
            in_copy(0, 0, c).start()

    @pl.when(b + 1 < B)
    def _():
        for c in range(_C):
            in_copy(b + 1, 1 - p, c).start()

    keyrow = key_ref[b][None, :].astype(jnp.bfloat16)
    onesrow = jnp.ones((1, W), dtype=jnp.bfloat16)
    for c in range(_C):
        in_copy(b, p, c).wait()
        mem_bf = mem_v[p, pl.ds(c * NC, NC), :].astype(jnp.bfloat16)
        dot_s[:, pl.ds(c * NC, NC)] = jax.lax.dot_general(
            keyrow, mem_bf, (((1,), (1,)), ((), ())),
            preferred_element_type=jnp.float32)
        sq = mem_bf * mem_bf
        msq_s[:, pl.ds(c * NC, NC)] = jax.lax.dot_general(
            onesrow, sq, (((1,), (1,)), ((), ())),
            preferred_element_type=jnp.float32)

    beta = beta_ref[b, 0]
    ag = ag_ref[b, 0]
    wg = wg_ref[b, 0]
    knorm = jnp.sqrt(jnp.sum(key_ref[b] * key_ref[b]))
    norm = knorm * jnp.sqrt(msq_s[:, :])
    score = beta * (dot_s[:, :] / (norm + _EPS))
    e = jnp.exp(score)
    inv_se = 1.0 / jnp.sum(e)

    prd = prd_ref[0]
    R = prd.shape[0]
    phi = 1.0 - fg_ref[b, 0] * prd[0:1]
    for r in range(1, R):
        phi = phi * (1.0 - fg_ref[b, r] * prd[r:r + 1])

    alloc = alloc_ref[0][None, :]
    wd = (wg * ag) * alloc + (wg * (1.0 - ag) * inv_se) * e

    lhs = jnp.concatenate([phi, -wd * phi, wd], axis=0)
    lhs_bf = lhs.astype(jnp.bfloat16)
    erase = erase_ref[b][None, :]
    wv = wv_ref[b][None, :]
    ones_f = jnp.ones((1, W), dtype=jnp.float32)
    zeros_f = jnp.zeros((1, W), dtype=jnp.float32)
    rhs = jnp.concatenate(
        [jnp.concatenate([ones_f, zeros_f], axis=1),
         jnp.concatenate([erase, zeros_f], axis=1),
         jnp.concatenate([zeros_f, wv], axis=1)], axis=0)
    rhs_bf = rhs.astype(jnp.bfloat16)

    @pl.when(b >= 2)
    def _():
        for c in range(_C):
            out_copy(b - 2, p, c).wait()

    for c in range(_C):
        m = jax.lax.dot_general(
            lhs_bf[:, c * NC:(c + 1) * NC], rhs_bf, (((0,), (0,)), ((), ())),
            preferred_element_type=jnp.float32)
        mem_c = mem_v[p, pl.ds(c * NC, NC), :]
        out_v[p, pl.ds(c * NC, NC), :] = mem_c * m[:, :W] + m[:, W:]
        out_copy(b, p, c).start()

    @pl.when(b == B - 1)
    def _():
        for c in range(_C):
            out_copy(b - 1, 1 - p, c).wait()
            out_copy(b, p, c).wait()


def kernel(memory, write_content_key, write_beta, erase_vector, write_vector,
           alloc_gate, write_gate, free_gates, prev_read_dist):
    B, N, W = memory.shape
    R = free_gates.shape[1]
    alloc = jnp.asarray(_alloc_const(N))[None, :]

    full = lambda a: pl.BlockSpec(a.shape, lambda b: (0,) * a.ndim)
    hbm = pl.BlockSpec(memory_space=pltpu.MemorySpace.HBM)
    return pl.pallas_call(
        _body,
        grid=(B,),
        in_specs=[
            hbm,
            full(write_content_key),
            full(write_beta),
            full(erase_vector),
            full(write_vector),
            full(alloc_gate),
            full(write_gate),
            full(free_gates),
            pl.BlockSpec((1, R, N), lambda b: (b, 0, 0)),
            full(alloc),
        ],
        out_specs=hbm,
        out_shape=jax.ShapeDtypeStruct((B, N, W), memory.dtype),
        scratch_shapes=[
            pltpu.MemorySpace.VMEM((2, N, W), jnp.float32),
            pltpu.MemorySpace.VMEM((2, N, W), jnp.float32),
            pltpu.MemorySpace.VMEM((1, N), jnp.float32),
            pltpu.MemorySpace.VMEM((1, N), jnp.float32),
            pltpu.SemaphoreType.DMA((2, _C)),
            pltpu.SemaphoreType.DMA((2, _C)),
        ],
    )(memory, write_content_key, write_beta, erase_vector, write_vector,
      alloc_gate, write_gate, free_gates, prev_read_dist, alloc)


# speedup vs baseline: 9.9484x; 1.1213x over previous
import numpy as np
import jax
import jax.numpy as jnp
from jax.experimental import pallas as pl
from jax.experimental.pallas import tpu as pltpu

_EPS = 1e-06
_C = 4


def _alloc_const(n):
    u = np.full((n,), _EPS, dtype=np.float32)
    u_prod = np.cumprod(u, dtype=np.float32)
    one_minus = (np.float32(1.0) - u).astype(np.float32)
    scores = np.concatenate([one_minus[:1], one_minus[1:] * u_prod[:-1]])
    return scores.astype(np.float32)


def _body(mem_hbm, key_ref, beta_ref, erase_ref, wv_ref, ag_ref, wg_ref,
          fg_ref, prd_ref, alloc_ref, out_hbm,
          mem_v, out_v, dot_s, msq_s, in_sem, out_sem):
    b = pl.program_id(0)
    B = pl.num_programs(0)
    N, W = mem_hbm.shape[1], mem_hbm.shape[2]
    NC = N // _C
    p = jax.lax.rem(b, 2)

    def in_copy(batch, par, c):
        return pltpu.make_async_copy(
            mem_hbm.at[batch, pl.ds(c * NC, NC), :],
            mem_v.at[par, pl.ds(c * NC, NC), :],
            in_sem.at[par, c])

    def out_copy(batch, par, c):
        return pltpu.make_async_copy(
            out_v.at[par, pl.ds(c * NC, NC), :],
            out_hbm.at[batch, pl.ds(c * NC, NC), :],
            out_sem.at[par, c])

    @pl.when(b == 0)
    def _():
        for c in range(_C):
            in_copy(0, 0, c).start()

    @pl.when(b + 1 < B)
    def _():
        for c in range(_C):
            in_copy(b + 1, 1 - p, c).start()

    keyrow = key_ref[b][None, :].astype(jnp.bfloat16)
    onesrow = jnp.ones((1, W), dtype=jnp.bfloat16)
    for c in range(_C):
        in_copy(b, p, c).wait()
        mem_bf = mem_v[p, pl.ds(c * NC, NC), :].astype(jnp.bfloat16)
        dot_s[:, pl.ds(c * NC, NC)] = jax.lax.dot_general(
            keyrow, mem_bf, (((1,), (1,)), ((), ())),
            preferred_element_type=jnp.float32)
        sq = mem_bf * mem_bf
        msq_s[:, pl.ds(c * NC, NC)] = jax.lax.dot_general(
            onesrow, sq, (((1,), (1,)), ((), ())),
            preferred_element_type=jnp.float32)

    beta = beta_ref[b, 0]
    ag = ag_ref[b, 0]
    wg = wg_ref[b, 0]
    knorm = jnp.sqrt(jnp.sum(key_ref[b] * key_ref[b]))
    norm = knorm * jnp.sqrt(msq_s[:, :])
    score = beta * (dot_s[:, :] / (norm + _EPS))
    e = jnp.exp(score)
    inv_se = 1.0 / jnp.sum(e)

    prd = prd_ref[0]
    R = prd.shape[0]
    phi = 1.0 - fg_ref[b, 0] * prd[0:1]
    for r in range(1, R):
        phi = phi * (1.0 - fg_ref[b, r] * prd[r:r + 1])

    alloc = alloc_ref[0][None, :]
    wd = (wg * ag) * alloc + (wg * (1.0 - ag) * inv_se) * e

    lhs = jnp.concatenate([phi, -wd * phi, wd], axis=0)
    lhs_bf = lhs.astype(jnp.bfloat16)
    erase = erase_ref[b][None, :]
    wv = wv_ref[b][None, :]
    ones_f = jnp.ones((1, W), dtype=jnp.float32)
    zeros_f = jnp.zeros((1, W), dtype=jnp.float32)
    rhs = jnp.concatenate(
        [jnp.concatenate([ones_f, zeros_f], axis=1),
         jnp.concatenate([erase, zeros_f], axis=1),
         jnp.concatenate([zeros_f, wv], axis=1)], axis=0)
    rhs_bf = rhs.astype(jnp.bfloat16)

    @pl.when(b >= 2)
    def _():
        for c in range(_C):
            out_copy(b - 2, p, c).wait()

    for c in range(_C):
        m = jax.lax.dot_general(
            lhs_bf[:, c * NC:(c + 1) * NC], rhs_bf, (((0,), (0,)), ((), ())),
            preferred_element_type=jnp.float32)
        mem_c = mem_v[p, pl.ds(c * NC, NC), :]
        out_v[p, pl.ds(c * NC, NC), :] = mem_c * m[:, :W] + m[:, W:]
        out_copy(b, p, c).start()

    @pl.when(b == B - 1)
    def _():
        for c in range(_C):
            out_copy(b - 1, 1 - p, c).wait()
            out_copy(b, p, c).wait()


def kernel(memory, write_content_key, write_beta, erase_vector, write_vector,
           alloc_gate, write_gate, free_gates, prev_read_dist):
    B, N, W = memory.shape
    R = free_gates.shape[1]
    alloc = jnp.asarray(_alloc_const(N))[None, :]

    full = lambda a: pl.BlockSpec(a.shape, lambda b: (0,) * a.ndim)
    hbm = pl.BlockSpec(memory_space=pltpu.MemorySpace.HBM)
    return pl.pallas_call(
        _body,
        grid=(B,),
        in_specs=[
            hbm,
            full(write_content_key),
            full(write_beta),
            full(erase_vector),
            full(write_vector),
            full(alloc_gate),
            full(write_gate),
            full(free_gates),
            pl.BlockSpec((1, R, N), lambda b: (b, 0, 0)),
            full(alloc),
        ],
        out_specs=hbm,
        out_shape=jax.ShapeDtypeStruct((B, N, W), memory.dtype),
        scratch_shapes=[
            pltpu.MemorySpace.VMEM((2, N, W), jnp.float32),
            pltpu.MemorySpace.VMEM((2, N, W), jnp.float32),
            pltpu.MemorySpace.VMEM((1, N), jnp.float32),
            pltpu.MemorySpace.VMEM((1, N), jnp.float32),
            pltpu.SemaphoreType.DMA((2, _C)),
            pltpu.SemaphoreType.DMA((2, _C)),
        ],
    )(memory, write_content_key, write_beta, erase_vector, write_vector,
      alloc_gate, write_gate, free_gates, prev_read_dist, alloc)


# speedup vs baseline: 10.1784x; 1.0231x over previous
import numpy as np
import jax
import jax.numpy as jnp
from jax.experimental import pallas as pl

_EPS = 1e-06


def _alloc_const(n):
    u = np.full((n,), _EPS, dtype=np.float32)
    u_prod = np.cumprod(u, dtype=np.float32)
    one_minus = (np.float32(1.0) - u).astype(np.float32)
    scores = np.concatenate([one_minus[:1], one_minus[1:] * u_prod[:-1]])
    return scores.astype(np.float32)


def _body(mem_ref, key_ref, beta_ref, erase_ref, wv_ref, ag_ref, wg_ref,
          fg_ref, prd_ref, alloc_ref, out_ref):
    b = pl.program_id(0)
    mem = mem_ref[0]
    N, W = mem.shape
    beta = beta_ref[b, 0]
    ag = ag_ref[b, 0]
    wg = wg_ref[b, 0]

    keyrow = key_ref[b][None, :]
    onesrow = jnp.ones((1, W), dtype=mem.dtype)
    dot2 = jax.lax.dot_general(keyrow, mem, (((1,), (1,)), ((), ())),
                               preferred_element_type=jnp.float32)
    sq = mem * mem
    msq2 = jax.lax.dot_general(onesrow, sq, (((1,), (1,)), ((), ())),
                               preferred_element_type=jnp.float32)

    knorm = jnp.sqrt(jnp.sum(key_ref[b] * key_ref[b]))
    norm = knorm * jnp.sqrt(msq2)
    score = beta * (dot2 / (norm + _EPS))
    e = jnp.exp(score)
    inv_se = 1.0 / jnp.sum(e)

    prd = prd_ref[0]
    R = prd.shape[0]
    phi = 1.0 - fg_ref[b, 0] * prd[0:1]
    for r in range(1, R):
        phi = phi * (1.0 - fg_ref[b, r] * prd[r:r + 1])

    alloc = alloc_ref[0][None, :]
    wd = (wg * ag) * alloc + (wg * (1.0 - ag) * inv_se) * e

    erase = erase_ref[b][None, :]
    wv = wv_ref[b][None, :]
    zerosrow = jnp.zeros((1, W), dtype=mem.dtype)
    lhs = jnp.concatenate([phi, -wd * phi, wd], axis=0)
    rhs = jnp.concatenate(
        [jnp.concatenate([onesrow, zerosrow], axis=1),
         jnp.concatenate([erase, zerosrow], axis=1),
         jnp.concatenate([zerosrow, wv], axis=1)], axis=0)
    m = jax.lax.dot_general(lhs, rhs, (((0,), (0,)), ((), ())),
                            preferred_element_type=jnp.float32)
    out_ref[0] = mem * m[:, :W] + m[:, W:]


def kernel(memory, write_content_key, write_beta, erase_vector, write_vector,
           alloc_gate, write_gate, free_gates, prev_read_dist):
    B, N, W = memory.shape
    R = free_gates.shape[1]
    alloc = jnp.asarray(_alloc_const(N))[None, :]

    full = lambda a: pl.BlockSpec(a.shape, lambda b: (0,) * a.ndim)
    return pl.pallas_call(
        _body,
        grid=(B,),
        in_specs=[
            pl.BlockSpec((1, N, W), lambda b: (b, 0, 0)),
            full(write_content_key),
            full(write_beta),
            full(erase_vector),
            full(write_vector),
            full(alloc_gate),
            full(write_gate),
            full(free_gates),
            pl.BlockSpec((1, R, N), lambda b: (b, 0, 0)),
            full(alloc),
        ],
        out_specs=pl.BlockSpec((1, N, W), lambda b: (b, 0, 0)),
        out_shape=jax.ShapeDtypeStruct((B, N, W), memory.dtype),
    )(memory, write_content_key, write_beta, erase_vector, write_vector,
      alloc_gate, write_gate, free_gates, prev_read_dist, alloc)
